# baseline (speedup 1.0000x reference)
import itertools

import jax
import jax.numpy as jnp
from jax.experimental import pallas as pl

_NUM_FIELDS = 26
_EMB = 4
_COMBOS = list(itertools.combinations(range(_NUM_FIELDS), 2))
_BATCH = 4096
_BLK = 512


def _pooled(table, idx):
    e = jnp.take(table, idx, axis=0)
    return jnp.mean(e, axis=1)


def _finish_body(a_ref, b_ref, l_ref, w_ref, bias_ref, o_ref):
    a = a_ref[...]
    b = b_ref[...]
    fm = jnp.sum(a * b, axis=1, keepdims=True)
    lin = jnp.sum(l_ref[...] * w_ref[...], axis=1, keepdims=True) + bias_ref[0, 0]
    o_ref[...] = jax.nn.sigmoid(jax.nn.relu(lin) + fm)


def kernel(params, field_0, field_1, field_2, field_3, field_4, field_5,
           field_6, field_7, field_8, field_9, field_10, field_11,
           field_12, field_13, field_14, field_15, field_16, field_17,
           field_18, field_19, field_20, field_21, field_22, field_23,
           field_24, field_25):
    fields = [field_0, field_1, field_2, field_3, field_4, field_5,
              field_6, field_7, field_8, field_9, field_10, field_11,
              field_12, field_13, field_14, field_15, field_16, field_17,
              field_18, field_19, field_20, field_21, field_22, field_23,
              field_24, field_25]
    ffm = params["ffm"]

    a_parts = []
    b_parts = []
    for i, j in _COMBOS:
        a_parts.append(_pooled(ffm[f"{i}-{j}"], fields[i]))
        b_parts.append(_pooled(ffm[f"{j}-{i}"], fields[j]))
    a = jnp.concatenate(a_parts, axis=1)
    b = jnp.concatenate(b_parts, axis=1)

    lin_parts = [_pooled(params["lin"][str(i)], fields[i])
                 for i in range(_NUM_FIELDS)]
    lin = jnp.concatenate(lin_parts, axis=1)

    w = params["W"][:, 0][None, :]
    bias = params["b"].reshape(1, 1)

    grid = (_BATCH // _BLK,)
    d = len(_COMBOS) * _EMB
    out = pl.pallas_call(
        _finish_body,
        grid=grid,
        in_specs=[
            pl.BlockSpec((_BLK, d), lambda g: (g, 0)),
            pl.BlockSpec((_BLK, d), lambda g: (g, 0)),
            pl.BlockSpec((_BLK, _NUM_FIELDS), lambda g: (g, 0)),
            pl.BlockSpec((1, _NUM_FIELDS), lambda g: (0, 0)),
            pl.BlockSpec((1, 1), lambda g: (0, 0)),
        ],
        out_specs=pl.BlockSpec((_BLK, 1), lambda g: (g, 0)),
        out_shape=jax.ShapeDtypeStruct((_BATCH, 1), jnp.float32),
    )(a, b, lin, w, bias)
    return out


# speedup vs baseline: 1.6451x; 1.6451x over previous
import functools
import itertools

import jax
import jax.numpy as jnp
from jax import lax
from jax.experimental import pallas as pl
from jax.experimental.pallas import tpu as pltpu
from jax.experimental.pallas import tpu_sc as plsc

_NUM_FIELDS = 26
_EMB = 4
_BATCH = 4096
_HIST = 20
_COMBOS = list(itertools.combinations(range(_NUM_FIELDS), 2))
_PAIR_IDX = {p: n for n, p in enumerate(_COMBOS)}
_NPAIR = len(_COMBOS)
_W = 112
_DIMS = [100000] * 6 + [1000] * 20
_ROW_OFF = [0] * _NUM_FIELDS
for _f in range(1, _NUM_FIELDS):
    _ROW_OFF[_f] = _ROW_OFF[_f - 1] + _DIMS[_f - 1]
_SEQ = (0, 1)
_NW = 32
_BPT = _BATCH // _NW
_IDX_ROWS_SEQ = _BPT * _HIST // 128
_SUB = 4
_ROWS_PER_SUB = 32 * _HIST
_AB_ROWS = 2 * _NPAIR * _EMB


def _others(f):
    return [j for j in range(_NUM_FIELDS) if j != f]


def _ab_row(f, j):
    jj_pair = (f, j) if f < j else (j, f)
    side = 0 if f < j else _NPAIR * _EMB
    return side + _EMB * _PAIR_IDX[jj_pair]


def _sc_body(mega, idx_hbm, ab, lin_out, idx_buf, rows_v, pooled, tr, sem_g,
             sem_w):
    w = lax.axis_index("s") * 2 + lax.axis_index("c")
    iota16 = lax.iota(jnp.int32, 16)
    pltpu.sync_copy(idx_hbm.at[pl.ds(w * 64, 64)], idx_buf)

    def transpose():
        def _tr_g(g, _):
            def _tr_f(ft, __):
                v = plsc.load_gather(
                    pooled, [g * 16 + iota16,
                             jnp.zeros((16,), jnp.int32) + ft])
                tr[ft, pl.ds(g * 16, 16)] = v
                return __
            return lax.fori_loop(0, _W, _tr_f, _)
        lax.fori_loop(0, 8, _tr_g, 0)

    def write_out(f):
        def _wr(jj, _):
            j = jj + (jj >= f).astype(jnp.int32)
            i_ = jnp.minimum(f, j)
            j_ = jnp.maximum(f, j)
            p = i_ * 25 - ((i_ * (i_ - 1)) >> 1) + (j_ - i_ - 1)
            r0 = jnp.where(f < j, 0, _NPAIR * _EMB) + _EMB * p
            pltpu.async_copy(
                tr.at[pl.ds(jj * _EMB, _EMB)],
                ab.at[pl.ds(r0, _EMB), pl.ds(w * _BPT, _BPT)], sem_w)
            return _
        lax.fori_loop(0, 25, _wr, 0)
        pltpu.async_copy(tr.at[pl.ds(100, 1)],
                         lin_out.at[pl.ds(f, 1), pl.ds(w * _BPT, _BPT)],
                         sem_w)
        for _ in range(25):
            pltpu.make_async_copy(ab.at[pl.ds(0, _EMB), pl.ds(0, _BPT)],
                                  tr.at[pl.ds(0, _EMB)], sem_w).wait()
        pltpu.make_async_copy(ab.at[pl.ds(0, 1), pl.ds(0, _BPT)],
                              tr.at[pl.ds(0, 1)], sem_w).wait()

    for f in _SEQ:
        def _sub(s, _, f=f):
            hs = []
            for j in range(5):
                hs.append(pltpu.async_copy(
                    mega.at[idx_buf.at[f * _IDX_ROWS_SEQ + s * 5 + j]],
                    rows_v.at[pl.ds(j * 128, 128)], sem_g))
            for h in hs:
                h.wait()

            def _pool(b, __):
                for v in range(_W // 16):
                    acc = jnp.zeros((16,), jnp.float32)
                    for t in range(_HIST):
                        acc = acc + rows_v[b * _HIST + t, pl.ds(v * 16, 16)]
                    pooled[s * 32 + b, pl.ds(v * 16, 16)] = acc * (1.0 / _HIST)
                return __
            return lax.fori_loop(0, 32, _pool, _)
        lax.fori_loop(0, _SUB, _sub, 0)
        transpose()
        write_out(jnp.int32(f))

    def _field(f, _):
        pltpu.async_copy(mega.at[idx_buf.at[38 + f]], pooled, sem_g).wait()
        transpose()
        write_out(f)
        return _
    lax.fori_loop(2, _NUM_FIELDS, _field, 0)


def _sc_gather(mega, idx):
    mesh = plsc.VectorSubcoreMesh(core_axis_name="c", subcore_axis_name="s")
    f = pl.kernel(
        _sc_body,
        mesh=mesh,
        compiler_params=pltpu.CompilerParams(use_tc_tiling_on_sc=False,
                                             needs_layout_passes=False),
        out_type=[
            jax.ShapeDtypeStruct((_AB_ROWS, _BATCH), jnp.float32),
            jax.ShapeDtypeStruct((_NUM_FIELDS, _BATCH), jnp.float32),
        ],
        scratch_types=[
            pltpu.VMEM((64, 128), jnp.int32),
            pltpu.VMEM((_ROWS_PER_SUB, _W), jnp.float32),
            pltpu.VMEM((_BPT, _W), jnp.float32),
            pltpu.VMEM((_W, _BPT), jnp.float32),
            pltpu.SemaphoreType.DMA,
            pltpu.SemaphoreType.DMA,
        ],
    )
    return f(mega, idx)


def _finish_body(ab_ref, l_ref, w_ref, bias_ref, o_ref):
    a = ab_ref[pl.ds(0, _NPAIR * _EMB), :]
    b = ab_ref[pl.ds(_NPAIR * _EMB, _NPAIR * _EMB), :]
    fm = jnp.sum(a * b, axis=0)
    lin = jnp.sum(l_ref[...] * w_ref[...], axis=0) + bias_ref[0, 0]
    o_ref[...] = jax.nn.sigmoid(jax.nn.relu(lin) + fm)[:, None]


def kernel(params, field_0, field_1, field_2, field_3, field_4, field_5,
           field_6, field_7, field_8, field_9, field_10, field_11,
           field_12, field_13, field_14, field_15, field_16, field_17,
           field_18, field_19, field_20, field_21, field_22, field_23,
           field_24, field_25):
    fields = [field_0, field_1, field_2, field_3, field_4, field_5,
              field_6, field_7, field_8, field_9, field_10, field_11,
              field_12, field_13, field_14, field_15, field_16, field_17,
              field_18, field_19, field_20, field_21, field_22, field_23,
              field_24, field_25]
    ffm = params["ffm"]

    wide = []
    for f in range(_NUM_FIELDS):
        cols = [ffm[f"{f}-{j}"] for j in _others(f)]
        cols.append(params["lin"][str(f)])
        cols.append(jnp.zeros((_DIMS[f], _W - 100 - 1), jnp.float32))
        wide.append(jnp.concatenate(cols, axis=1))
    mega = jnp.concatenate(wide, axis=0)

    parts = [(fields[f].astype(jnp.int32) + _ROW_OFF[f])
             .reshape(_NW, _IDX_ROWS_SEQ, 128) for f in _SEQ]
    parts += [(fields[f][:, 0].astype(jnp.int32) + _ROW_OFF[f])
              .reshape(_NW, 1, 128) for f in range(2, _NUM_FIELDS)]
    idx = jnp.concatenate(parts, axis=1).reshape(-1, 128)

    ab, lin_feat = _sc_gather(mega, idx)

    w = params["W"]
    bias = params["b"].reshape(1, 1)
    blk = 512
    out = pl.pallas_call(
        _finish_body,
        grid=(_BATCH // blk,),
        in_specs=[
            pl.BlockSpec((_AB_ROWS, blk), lambda g: (0, g)),
            pl.BlockSpec((_NUM_FIELDS, blk), lambda g: (0, g)),
            pl.BlockSpec((_NUM_FIELDS, 1), lambda g: (0, 0)),
            pl.BlockSpec((1, 1), lambda g: (0, 0)),
        ],
        out_specs=pl.BlockSpec((blk, 1), lambda g: (g, 0)),
        out_shape=jax.ShapeDtypeStruct((_BATCH, 1), jnp.float32),
    )(ab, lin_feat, w, bias)
    return out


# speedup vs baseline: 3.5254x; 2.1429x over previous
import functools
import itertools

import jax
import jax.numpy as jnp
from jax import lax
from jax.experimental import pallas as pl
from jax.experimental.pallas import tpu as pltpu
from jax.experimental.pallas import tpu_sc as plsc

_NUM_FIELDS = 26
_EMB = 4
_BATCH = 4096
_HIST = 20
_COMBOS = list(itertools.combinations(range(_NUM_FIELDS), 2))
_PAIR_IDX = {p: n for n, p in enumerate(_COMBOS)}
_NPAIR = len(_COMBOS)
_W = 112
_DIMS = [100000] * 6 + [1000] * 20
_ROW_OFF = [0] * _NUM_FIELDS
for _f in range(1, _NUM_FIELDS):
    _ROW_OFF[_f] = _ROW_OFF[_f - 1] + _DIMS[_f - 1]
_SEQ = (0, 1)
_NW = 32
_BPT = _BATCH // _NW
_IDX_ROWS_SEQ = _BPT * _HIST // 128
_SUB = 4
_ROWS_PER_SUB = 32 * _HIST
_AB_ROWS = 2 * _NPAIR * _EMB


def _others(f):
    return [j for j in range(_NUM_FIELDS) if j != f]


def _ab_row(f, j):
    jj_pair = (f, j) if f < j else (j, f)
    side = 0 if f < j else _NPAIR * _EMB
    return side + _EMB * _PAIR_IDX[jj_pair]


def _sc_body(mega, idx_hbm, ab, lin_out, idx_buf, rows_v, pooled, tr, sem_g,
             sem_w):
    w = lax.axis_index("s") * 2 + lax.axis_index("c")
    iota16 = lax.iota(jnp.int32, 16)
    pltpu.sync_copy(idx_hbm.at[pl.ds(w * 64, 64)], idx_buf)

    def transpose():
        def _tr_g(g, _):
            def _tr_f(ft, __):
                v = plsc.load_gather(
                    pooled, [g * 16 + iota16,
                             jnp.zeros((16,), jnp.int32) + ft])
                tr[ft, pl.ds(g * 16, 16)] = v
                return __
            return lax.fori_loop(0, _W, _tr_f, _)
        lax.fori_loop(0, 8, _tr_g, 0)

    def write_out(f):
        def _wr(jj, _):
            j = jj + (jj >= f).astype(jnp.int32)
            i_ = jnp.minimum(f, j)
            j_ = jnp.maximum(f, j)
            p = i_ * 25 - ((i_ * (i_ - 1)) >> 1) + (j_ - i_ - 1)
            r0 = jnp.where(f < j, 0, _NPAIR * _EMB) + _EMB * p
            pltpu.async_copy(
                tr.at[pl.ds(jj * _EMB, _EMB)],
                ab.at[pl.ds(r0, _EMB), pl.ds(w * _BPT, _BPT)], sem_w)
            return _
        lax.fori_loop(0, 25, _wr, 0)
        pltpu.async_copy(tr.at[pl.ds(100, 1)],
                         lin_out.at[pl.ds(f, 1), pl.ds(w * _BPT, _BPT)],
                         sem_w)
        for _ in range(25):
            pltpu.make_async_copy(ab.at[pl.ds(0, _EMB), pl.ds(0, _BPT)],
                                  tr.at[pl.ds(0, _EMB)], sem_w).wait()
        pltpu.make_async_copy(ab.at[pl.ds(0, 1), pl.ds(0, _BPT)],
                              tr.at[pl.ds(0, 1)], sem_w).wait()

    for f in _SEQ:
        def _sub(s, _, f=f):
            hs = []
            for j in range(5):
                hs.append(pltpu.async_copy(
                    mega.at[idx_buf.at[f * _IDX_ROWS_SEQ + s * 5 + j]],
                    rows_v.at[pl.ds(j * 128, 128)], sem_g))
            for h in hs:
                h.wait()

            def _pool(b, __):
                for v in range(_W // 16):
                    acc = jnp.zeros((16,), jnp.float32)
                    for t in range(_HIST):
                        acc = acc + rows_v[b * _HIST + t, pl.ds(v * 16, 16)]
                    pooled[s * 32 + b, pl.ds(v * 16, 16)] = acc * (1.0 / _HIST)
                return __
            return lax.fori_loop(0, 32, _pool, _)
        lax.fori_loop(0, _SUB, _sub, 0)
        transpose()
        write_out(jnp.int32(f))

    def _field(f, _):
        pltpu.async_copy(mega.at[idx_buf.at[38 + f]], pooled, sem_g).wait()
        transpose()
        write_out(f)
        return _
    lax.fori_loop(2, _NUM_FIELDS, _field, 0)


def _sc_gather(mega, idx):
    mesh = plsc.VectorSubcoreMesh(core_axis_name="c", subcore_axis_name="s")
    f = pl.kernel(
        _sc_body,
        mesh=mesh,
        compiler_params=pltpu.CompilerParams(use_tc_tiling_on_sc=False,
                                             needs_layout_passes=False),
        out_type=[
            jax.ShapeDtypeStruct((_AB_ROWS, _BATCH), jnp.float32),
            jax.ShapeDtypeStruct((_NUM_FIELDS, _BATCH), jnp.float32),
        ],
        scratch_types=[
            pltpu.VMEM((64, 128), jnp.int32),
            pltpu.VMEM((_ROWS_PER_SUB, _W), jnp.float32),
            pltpu.VMEM((_BPT, _W), jnp.float32),
            pltpu.VMEM((_W, _BPT), jnp.float32),
            pltpu.SemaphoreType.DMA,
            pltpu.SemaphoreType.DMA,
        ],
    )
    return f(mega, idx)


def _finish_body(ab_ref, l_ref, w_ref, bias_ref, o_ref):
    a = ab_ref[pl.ds(0, _NPAIR * _EMB), :]
    b = ab_ref[pl.ds(_NPAIR * _EMB, _NPAIR * _EMB), :]
    fm = jnp.sum(a * b, axis=0)
    lin = jnp.sum(l_ref[...] * w_ref[...], axis=0) + bias_ref[0, 0]
    o_ref[...] = jax.nn.sigmoid(jax.nn.relu(lin) + fm)[:, None]


def kernel(params, field_0, field_1, field_2, field_3, field_4, field_5,
           field_6, field_7, field_8, field_9, field_10, field_11,
           field_12, field_13, field_14, field_15, field_16, field_17,
           field_18, field_19, field_20, field_21, field_22, field_23,
           field_24, field_25):
    fields = [field_0, field_1, field_2, field_3, field_4, field_5,
              field_6, field_7, field_8, field_9, field_10, field_11,
              field_12, field_13, field_14, field_15, field_16, field_17,
              field_18, field_19, field_20, field_21, field_22, field_23,
              field_24, field_25]
    ffm = params["ffm"]

    nrows = sum(_DIMS)
    slabs = [jnp.concatenate([ffm[f"{f}-{_others(f)[k]}"]
                              for f in range(_NUM_FIELDS)], axis=0)
             for k in range(25)]
    lin_slab = jnp.concatenate([params["lin"][str(f)]
                                for f in range(_NUM_FIELDS)], axis=0)
    mega = jnp.concatenate(
        [jnp.stack(slabs, axis=1).reshape(nrows, 100),
         lin_slab,
         jnp.zeros((nrows, _W - 101), jnp.float32)], axis=1)

    parts = [(fields[f].astype(jnp.int32) + _ROW_OFF[f])
             .reshape(_NW, _IDX_ROWS_SEQ, 128) for f in _SEQ]
    parts += [(fields[f][:, 0].astype(jnp.int32) + _ROW_OFF[f])
              .reshape(_NW, 1, 128) for f in range(2, _NUM_FIELDS)]
    idx = jnp.concatenate(parts, axis=1).reshape(-1, 128)

    ab, lin_feat = _sc_gather(mega, idx)

    w = params["W"]
    bias = params["b"].reshape(1, 1)
    blk = 512
    out = pl.pallas_call(
        _finish_body,
        grid=(_BATCH // blk,),
        in_specs=[
            pl.BlockSpec((_AB_ROWS, blk), lambda g: (0, g)),
            pl.BlockSpec((_NUM_FIELDS, blk), lambda g: (0, g)),
            pl.BlockSpec((_NUM_FIELDS, 1), lambda g: (0, 0)),
            pl.BlockSpec((1, 1), lambda g: (0, 0)),
        ],
        out_specs=pl.BlockSpec((blk, 1), lambda g: (g, 0)),
        out_shape=jax.ShapeDtypeStruct((_BATCH, 1), jnp.float32),
    )(ab, lin_feat, w, bias)
    return out


# speedup vs baseline: 3.5264x; 1.0003x over previous
import functools
import itertools

import jax
import jax.numpy as jnp
from jax import lax
from jax.experimental import pallas as pl
from jax.experimental.pallas import tpu as pltpu
from jax.experimental.pallas import tpu_sc as plsc

_NUM_FIELDS = 26
_EMB = 4
_BATCH = 4096
_HIST = 20
_COMBOS = list(itertools.combinations(range(_NUM_FIELDS), 2))
_PAIR_IDX = {p: n for n, p in enumerate(_COMBOS)}
_NPAIR = len(_COMBOS)
_W = 112
_DIMS = [100000] * 6 + [1000] * 20
_ROW_OFF = [0] * _NUM_FIELDS
for _f in range(1, _NUM_FIELDS):
    _ROW_OFF[_f] = _ROW_OFF[_f - 1] + _DIMS[_f - 1]
_SEQ = (0, 1)
_NW = 32
_BPT = _BATCH // _NW
_IDX_ROWS_SEQ = _BPT * _HIST // 128
_SUB = 4
_ROWS_PER_SUB = 32 * _HIST
_AB_ROWS = 2 * _NPAIR * _EMB


def _others(f):
    return [j for j in range(_NUM_FIELDS) if j != f]


def _ab_row(f, j):
    jj_pair = (f, j) if f < j else (j, f)
    side = 0 if f < j else _NPAIR * _EMB
    return side + _EMB * _PAIR_IDX[jj_pair]


def _sc_body(mega, idx_hbm, ab, lin_out, idx_buf, rows_v, pooled, tr, sem_g,
             sem_w):
    w = lax.axis_index("s") * 2 + lax.axis_index("c")
    iota16 = lax.iota(jnp.int32, 16)
    pltpu.sync_copy(idx_hbm.at[pl.ds(w * 64, 64)], idx_buf)

    def transpose():
        def _tr_g(g, _):
            def _tr_f(ft, __):
                v = plsc.load_gather(
                    pooled, [g * 16 + iota16,
                             jnp.zeros((16,), jnp.int32) + ft])
                tr[ft, pl.ds(g * 16, 16)] = v
                return __
            return lax.fori_loop(0, _W, _tr_f, _)
        lax.fori_loop(0, 8, _tr_g, 0)

    def write_out(f):
        def _wr(jj, _):
            j = jj + (jj >= f).astype(jnp.int32)
            i_ = jnp.minimum(f, j)
            j_ = jnp.maximum(f, j)
            p = i_ * 25 - lax.shift_right_logical(i_ * (i_ - 1), 1) \
                + (j_ - i_ - 1)
            r0 = jnp.where(f < j, 0, _NPAIR * _EMB) + _EMB * p
            pltpu.async_copy(
                tr.at[pl.ds(jj * _EMB, _EMB)],
                ab.at[pl.ds(r0, _EMB), pl.ds(w * _BPT, _BPT)], sem_w)
            return _
        lax.fori_loop(0, 25, _wr, 0)
        pltpu.async_copy(tr.at[pl.ds(100, 1)],
                         lin_out.at[pl.ds(f, 1), pl.ds(w * _BPT, _BPT)],
                         sem_w)
        for _ in range(25):
            pltpu.make_async_copy(ab.at[pl.ds(0, _EMB), pl.ds(0, _BPT)],
                                  tr.at[pl.ds(0, _EMB)], sem_w).wait()
        pltpu.make_async_copy(ab.at[pl.ds(0, 1), pl.ds(0, _BPT)],
                              tr.at[pl.ds(0, 1)], sem_w).wait()

    for f in _SEQ:
        def _sub(s, _, f=f):
            hs = []
            for j in range(5):
                hs.append(pltpu.async_copy(
                    mega.at[idx_buf.at[f * _IDX_ROWS_SEQ + s * 5 + j]],
                    rows_v.at[pl.ds(j * 128, 128)], sem_g))
            for h in hs:
                h.wait()

            def _pool(b, __):
                for v in range(_W // 16):
                    acc = jnp.zeros((16,), jnp.float32)
                    for t in range(_HIST):
                        acc = acc + rows_v[b * _HIST + t, pl.ds(v * 16, 16)]
                    pooled[s * 32 + b, pl.ds(v * 16, 16)] = acc * (1.0 / _HIST)
                return __
            return lax.fori_loop(0, 32, _pool, _)
        lax.fori_loop(0, _SUB, _sub, 0)
        transpose()
        write_out(jnp.int32(f))

    def _field(f, _):
        pltpu.async_copy(mega.at[idx_buf.at[38 + f]], pooled, sem_g).wait()
        transpose()
        write_out(f)
        return _
    lax.fori_loop(2, _NUM_FIELDS, _field, 0)


def _sc_gather(mega, idx):
    mesh = plsc.VectorSubcoreMesh(core_axis_name="c", subcore_axis_name="s")
    f = pl.kernel(
        _sc_body,
        mesh=mesh,
        compiler_params=pltpu.CompilerParams(use_tc_tiling_on_sc=False,
                                             needs_layout_passes=False),
        out_type=[
            jax.ShapeDtypeStruct((_AB_ROWS, _BATCH), jnp.float32),
            jax.ShapeDtypeStruct((_NUM_FIELDS, _BATCH), jnp.float32),
        ],
        scratch_types=[
            pltpu.VMEM((64, 128), jnp.int32),
            pltpu.VMEM((_ROWS_PER_SUB, _W), jnp.float32),
            pltpu.VMEM((_BPT, _W), jnp.float32),
            pltpu.VMEM((_W, _BPT), jnp.float32),
            pltpu.SemaphoreType.DMA,
            pltpu.SemaphoreType.DMA,
        ],
    )
    return f(mega, idx)


def _finish_body(ab_ref, l_ref, w_ref, bias_ref, o_ref):
    a = ab_ref[pl.ds(0, _NPAIR * _EMB), :]
    b = ab_ref[pl.ds(_NPAIR * _EMB, _NPAIR * _EMB), :]
    fm = jnp.sum(a * b, axis=0)
    lin = jnp.sum(l_ref[...] * w_ref[...], axis=0) + bias_ref[0, 0]
    o_ref[...] = jax.nn.sigmoid(jax.nn.relu(lin) + fm)[:, None]


def kernel(params, field_0, field_1, field_2, field_3, field_4, field_5,
           field_6, field_7, field_8, field_9, field_10, field_11,
           field_12, field_13, field_14, field_15, field_16, field_17,
           field_18, field_19, field_20, field_21, field_22, field_23,
           field_24, field_25):
    fields = [field_0, field_1, field_2, field_3, field_4, field_5,
              field_6, field_7, field_8, field_9, field_10, field_11,
              field_12, field_13, field_14, field_15, field_16, field_17,
              field_18, field_19, field_20, field_21, field_22, field_23,
              field_24, field_25]
    ffm = params["ffm"]

    nrows = sum(_DIMS)
    slabs = [jnp.concatenate([ffm[f"{f}-{_others(f)[k]}"]
                              for f in range(_NUM_FIELDS)], axis=0)
             for k in range(25)]
    lin_slab = jnp.concatenate([params["lin"][str(f)]
                                for f in range(_NUM_FIELDS)], axis=0)
    mega = jnp.concatenate(
        [jnp.stack(slabs, axis=1).reshape(nrows, 100),
         lin_slab,
         jnp.zeros((nrows, _W - 101), jnp.float32)], axis=1)

    parts = [(fields[f].astype(jnp.int32) + _ROW_OFF[f])
             .reshape(_NW, _IDX_ROWS_SEQ, 128) for f in _SEQ]
    parts += [(fields[f][:, 0].astype(jnp.int32) + _ROW_OFF[f])
              .reshape(_NW, 1, 128) for f in range(2, _NUM_FIELDS)]
    idx = jnp.concatenate(parts, axis=1).reshape(-1, 128)

    ab, lin_feat = _sc_gather(mega, idx)

    w = params["W"]
    bias = params["b"].reshape(1, 1)
    blk = 512
    out = pl.pallas_call(
        _finish_body,
        grid=(_BATCH // blk,),
        in_specs=[
            pl.BlockSpec((_AB_ROWS, blk), lambda g: (0, g)),
            pl.BlockSpec((_NUM_FIELDS, blk), lambda g: (0, g)),
            pl.BlockSpec((_NUM_FIELDS, 1), lambda g: (0, 0)),
            pl.BlockSpec((1, 1), lambda g: (0, 0)),
        ],
        out_specs=pl.BlockSpec((blk, 1), lambda g: (g, 0)),
        out_shape=jax.ShapeDtypeStruct((_BATCH, 1), jnp.float32),
    )(ab, lin_feat, w, bias)
    return out


# speedup vs baseline: 3.5983x; 1.0204x over previous
import functools
import itertools

import jax
import jax.numpy as jnp
from jax import lax
from jax.experimental import pallas as pl
from jax.experimental.pallas import tpu as pltpu
from jax.experimental.pallas import tpu_sc as plsc

_NUM_FIELDS = 26
_EMB = 4
_BATCH = 4096
_HIST = 20
_COMBOS = list(itertools.combinations(range(_NUM_FIELDS), 2))
_PAIR_IDX = {p: n for n, p in enumerate(_COMBOS)}
_NPAIR = len(_COMBOS)
_W = 112
_DIMS = [100000] * 6 + [1000] * 20
_ROW_OFF = [0] * _NUM_FIELDS
for _f in range(1, _NUM_FIELDS):
    _ROW_OFF[_f] = _ROW_OFF[_f - 1] + _DIMS[_f - 1]
_SEQ = (0, 1)
_NW = 32
_BPT = _BATCH // _NW
_IDX_ROWS_SEQ = _BPT * _HIST // 128
_SUB = 4
_ROWS_PER_SUB = 32 * _HIST
_AB_ROWS = 2 * _NPAIR * _EMB


def _others(f):
    return [j for j in range(_NUM_FIELDS) if j != f]


def _ab_row(f, j):
    jj_pair = (f, j) if f < j else (j, f)
    side = 0 if f < j else _NPAIR * _EMB
    return side + _EMB * _PAIR_IDX[jj_pair]


def _sc_body(mega, idx_hbm, ab, lin_out, idx_buf, rows_v, pooled, tr, sem_g,
             sem_w):
    w = lax.axis_index("s") * 2 + lax.axis_index("c")
    iota16 = lax.iota(jnp.int32, 16)
    pltpu.sync_copy(idx_hbm.at[pl.ds(w * 64, 64)], idx_buf)

    def transpose():
        def _tr_g(g, _):
            def _tr_f(ft, __):
                v = plsc.load_gather(
                    pooled, [g * 16 + iota16,
                             jnp.zeros((16,), jnp.int32) + ft])
                tr[ft, pl.ds(g * 16, 16)] = v
                return __
            return lax.fori_loop(0, _W, _tr_f, _)
        lax.fori_loop(0, 8, _tr_g, 0)

    def write_out(f):
        def _wr(jj, _):
            j = jj + (jj >= f).astype(jnp.int32)
            i_ = jnp.minimum(f, j)
            j_ = jnp.maximum(f, j)
            p = i_ * 25 - lax.shift_right_logical(i_ * (i_ - 1), 1) \
                + (j_ - i_ - 1)
            r0 = jnp.where(f < j, 0, _NPAIR * _EMB) + _EMB * p
            pltpu.async_copy(
                tr.at[pl.ds(jj * _EMB, _EMB)],
                ab.at[pl.ds(r0, _EMB), pl.ds(w * _BPT, _BPT)], sem_w)
            return _
        lax.fori_loop(0, 25, _wr, 0)
        pltpu.async_copy(tr.at[pl.ds(100, 1)],
                         lin_out.at[pl.ds(f, 1), pl.ds(w * _BPT, _BPT)],
                         sem_w)
        for _ in range(25):
            pltpu.make_async_copy(ab.at[pl.ds(0, _EMB), pl.ds(0, _BPT)],
                                  tr.at[pl.ds(0, _EMB)], sem_w).wait()
        pltpu.make_async_copy(ab.at[pl.ds(0, 1), pl.ds(0, _BPT)],
                              tr.at[pl.ds(0, 1)], sem_w).wait()

    for f in _SEQ:
        def _sub(s, _, f=f):
            hs = []
            for j in range(5):
                hs.append(pltpu.async_copy(
                    mega.at[idx_buf.at[f * _IDX_ROWS_SEQ + s * 5 + j]],
                    rows_v.at[pl.ds(j * 128, 128)], sem_g))
            for h in hs:
                h.wait()

            def _pool(b, __):
                for v in range(_W // 16):
                    acc = jnp.zeros((16,), jnp.float32)
                    for t in range(_HIST):
                        acc = acc + rows_v[b * _HIST + t, pl.ds(v * 16, 16)]
                    pooled[s * 32 + b, pl.ds(v * 16, 16)] = acc * (1.0 / _HIST)
                return __
            return lax.fori_loop(0, 32, _pool, _)
        lax.fori_loop(0, _SUB, _sub, 0)
        transpose()
        write_out(jnp.int32(f))

    def _field(f, _):
        pltpu.async_copy(mega.at[idx_buf.at[38 + f]], pooled, sem_g).wait()
        transpose()
        write_out(f)
        return _
    lax.fori_loop(2, _NUM_FIELDS, _field, 0)


def _sc_gather(mega, idx):
    mesh = plsc.VectorSubcoreMesh(core_axis_name="c", subcore_axis_name="s")
    f = pl.kernel(
        _sc_body,
        mesh=mesh,
        compiler_params=pltpu.CompilerParams(use_tc_tiling_on_sc=False,
                                             needs_layout_passes=False),
        out_type=[
            jax.ShapeDtypeStruct((_AB_ROWS, _BATCH), jnp.float32),
            jax.ShapeDtypeStruct((_NUM_FIELDS, _BATCH), jnp.float32),
        ],
        scratch_types=[
            pltpu.VMEM((64, 128), jnp.int32),
            pltpu.VMEM((_ROWS_PER_SUB, _W), jnp.float32),
            pltpu.VMEM((_BPT, _W), jnp.float32),
            pltpu.VMEM((_W, _BPT), jnp.float32),
            pltpu.SemaphoreType.DMA,
            pltpu.SemaphoreType.DMA,
        ],
    )
    return f(mega, idx)


def _finish_body(ab_ref, l_ref, w_ref, bias_ref, o_ref):
    a = ab_ref[pl.ds(0, _NPAIR * _EMB), :]
    b = ab_ref[pl.ds(_NPAIR * _EMB, _NPAIR * _EMB), :]
    fm = jnp.sum(a * b, axis=0)
    lin = jnp.sum(l_ref[...] * w_ref[...], axis=0) + bias_ref[0, 0]
    o_ref[...] = jax.nn.sigmoid(jax.nn.relu(lin) + fm)[:, None]


def kernel(params, field_0, field_1, field_2, field_3, field_4, field_5,
           field_6, field_7, field_8, field_9, field_10, field_11,
           field_12, field_13, field_14, field_15, field_16, field_17,
           field_18, field_19, field_20, field_21, field_22, field_23,
           field_24, field_25):
    fields = [field_0, field_1, field_2, field_3, field_4, field_5,
              field_6, field_7, field_8, field_9, field_10, field_11,
              field_12, field_13, field_14, field_15, field_16, field_17,
              field_18, field_19, field_20, field_21, field_22, field_23,
              field_24, field_25]
    ffm = params["ffm"]

    nrows = sum(_DIMS)
    slabs = [jnp.concatenate([ffm[f"{f}-{_others(f)[k]}"]
                              for f in range(_NUM_FIELDS)], axis=0)
             for k in range(25)]
    lin_slab = jnp.concatenate([params["lin"][str(f)]
                                for f in range(_NUM_FIELDS)], axis=0)
    lin4 = jnp.pad(lin_slab, ((0, 0), (0, 3)))
    zero_slab = jnp.zeros((nrows, _EMB), jnp.float32)
    st = jnp.stack(slabs + [lin4, zero_slab, zero_slab], axis=0)
    mega = jnp.transpose(st, (1, 0, 2)).reshape(nrows, _W)

    parts = [(fields[f].astype(jnp.int32) + _ROW_OFF[f])
             .reshape(_NW, _IDX_ROWS_SEQ, 128) for f in _SEQ]
    parts += [(fields[f][:, 0].astype(jnp.int32) + _ROW_OFF[f])
              .reshape(_NW, 1, 128) for f in range(2, _NUM_FIELDS)]
    idx = jnp.concatenate(parts, axis=1).reshape(-1, 128)

    ab, lin_feat = _sc_gather(mega, idx)

    w = params["W"]
    bias = params["b"].reshape(1, 1)
    blk = 512
    out = pl.pallas_call(
        _finish_body,
        grid=(_BATCH // blk,),
        in_specs=[
            pl.BlockSpec((_AB_ROWS, blk), lambda g: (0, g)),
            pl.BlockSpec((_NUM_FIELDS, blk), lambda g: (0, g)),
            pl.BlockSpec((_NUM_FIELDS, 1), lambda g: (0, 0)),
            pl.BlockSpec((1, 1), lambda g: (0, 0)),
        ],
        out_specs=pl.BlockSpec((blk, 1), lambda g: (g, 0)),
        out_shape=jax.ShapeDtypeStruct((_BATCH, 1), jnp.float32),
    )(ab, lin_feat, w, bias)
    return out
